# baseline (speedup 1.0000x reference)
import functools

import jax
import jax.numpy as jnp
from jax import lax
from jax.experimental import pallas as pl
from jax.experimental.pallas import tpu as pltpu
from jax.experimental.pallas import tpu_sc as plsc

N_NODES = 10000
N_EDGES = 320000
D = 128

NC = 2
NS = 16
NW = NC * NS

EDGES_PER_W = N_EDGES // NW
CHUNK = 50
NCHUNK = EDGES_PER_W // CHUNK
SBC = 5
NSB = NCHUNK // SBC
NBUF = 5
LEAD = 3
ROWS_PER_TILE = 624
REM_ROW0 = NS * ROWS_PER_TILE
REM_ROWS = N_NODES - REM_ROW0


def _sc_aggregate(x, src, dst):
    mesh = plsc.VectorSubcoreMesh(core_axis_name="c", subcore_axis_name="s")

    @functools.partial(
        pl.kernel,
        mesh=mesh,
        out_type=jax.ShapeDtypeStruct((NC, N_NODES, D), jnp.float32),
        scratch_types=[
            pltpu.VMEM((3, SBC, CHUNK), jnp.int32),
            pltpu.VMEM((3, SBC, CHUNK), jnp.int32),
            pltpu.VMEM((NBUF, CHUNK, D), jnp.float32),
            pltpu.VMEM_SHARED((N_NODES, D), jnp.float32),
            pltpu.SemaphoreType.DMA,
            pltpu.SemaphoreType.DMA,
            pltpu.SemaphoreType.DMA((NBUF,)),
        ],
    )
    def agg_kernel(x_hbm, src_hbm, dst_hbm, out_hbm, src_v, dst_v, rows_v, acc,
                   sem, isem, ssem):
        cid = lax.axis_index("c")
        sid = lax.axis_index("s")
        wid = cid * NS + sid

        def issue_idx_load(s, buf):
            pltpu.async_copy(src_hbm.at[wid, s], src_v.at[buf], isem)
            pltpu.async_copy(dst_hbm.at[wid, s], dst_v.at[buf], isem)

        def wait_idx_load(s, buf):
            pltpu.make_async_copy(src_hbm.at[wid, s], src_v.at[buf], isem).wait()
            pltpu.make_async_copy(dst_hbm.at[wid, s], dst_v.at[buf], isem).wait()

        def issue_gather(sbuf, row, b):
            pltpu.async_copy(x_hbm.at[src_v.at[sbuf].at[row]], rows_v.at[b], sem)

        issue_idx_load(0, 0)
        issue_idx_load(1, 1)

        zeros16 = jnp.zeros((16,), jnp.float32)
        zbuf = rows_v.at[NBUF - 1]

        def zero_row(r, _):
            for c in range(D // 16):
                zbuf[r, pl.ds(c * 16, 16)] = zeros16
            return 0

        lax.fori_loop(0, CHUNK, zero_row, 0)

        wait_idx_load(0, 0)
        for i in range(LEAD):
            issue_gather(0, i, i)

        row0 = sid * ROWS_PER_TILE
        nfull, nrem = divmod(ROWS_PER_TILE, CHUNK)
        for k in range(nfull):
            pltpu.sync_copy(zbuf, acc.at[pl.ds(row0 + k * CHUNK, CHUNK)])
        if nrem:
            pltpu.sync_copy(zbuf.at[pl.ds(0, nrem)],
                            acc.at[pl.ds(row0 + nfull * CHUNK, nrem)])

        @pl.when(sid == NS - 1)
        def _zero_rem():
            pltpu.sync_copy(zbuf.at[pl.ds(0, REM_ROWS)],
                            acc.at[pl.ds(REM_ROW0, REM_ROWS)])

        plsc.subcore_barrier()

        def sb_body(s, _):
            bsb = lax.rem(s, 3)
            bsb_next = lax.rem(s + 1, 3)
            for jj in range(SBC):
                j = s * SBC + jj
                b = lax.rem(j, NBUF)
                pltpu.make_async_copy(x_hbm.at[src_v.at[bsb].at[jj]],
                                      rows_v.at[b], sem).wait()
                pltpu.async_copy(rows_v.at[b], acc.at[dst_v.at[bsb].at[jj]],
                                 ssem.at[b], add=True)

                if jj == SBC - LEAD:
                    @pl.when(s + 1 < NSB)
                    def _wait_next_idx():
                        wait_idx_load(s + 1, bsb_next)

                bnext = lax.rem(j + LEAD, NBUF)

                @pl.when(j + LEAD < NCHUNK)
                def _lookahead_gather():
                    @pl.when(j >= NBUF - LEAD)
                    def _free_buf():
                        pltpu.make_async_copy(
                            rows_v.at[bnext], acc.at[dst_v.at[bsb].at[jj]],
                            ssem.at[bnext]).wait()
                    if jj + LEAD < SBC:
                        issue_gather(bsb, jj + LEAD, bnext)
                    else:
                        issue_gather(bsb_next, jj + LEAD - SBC, bnext)

            @pl.when(s + 2 < NSB)
            def _prefetch_idx():
                issue_idx_load(s + 2, lax.rem(s + 2, 3))

            return 0

        lax.fori_loop(0, NSB, sb_body, 0)

        for b in range(NBUF):
            pltpu.make_async_copy(rows_v.at[b], acc.at[dst_v.at[0].at[0]],
                                  ssem.at[b]).wait()
        plsc.subcore_barrier()

        pltpu.sync_copy(acc.at[pl.ds(row0, ROWS_PER_TILE)],
                        out_hbm.at[cid, pl.ds(row0, ROWS_PER_TILE)])

        @pl.when(sid == NS - 1)
        def _write_rem():
            pltpu.sync_copy(acc.at[pl.ds(REM_ROW0, REM_ROWS)],
                            out_hbm.at[cid, pl.ds(REM_ROW0, REM_ROWS)])

    return agg_kernel(x, src, dst)


def _tc_mlp(x, parts, W, b, relu):
    d_out = W.shape[1]
    block = 1000
    grid = N_NODES // block

    def mlp_kernel(x_ref, p_ref, w_ref, b_ref, o_ref):
        h = x_ref[...] + p_ref[0] + p_ref[1]
        y = jnp.dot(h, w_ref[...], preferred_element_type=jnp.float32) + b_ref[...]
        if relu:
            y = jnp.maximum(y, 0.0)
        o_ref[...] = y

    return pl.pallas_call(
        mlp_kernel,
        grid=(grid,),
        in_specs=[
            pl.BlockSpec((block, D), lambda i: (i, 0)),
            pl.BlockSpec((NC, block, D), lambda i: (0, i, 0)),
            pl.BlockSpec((D, d_out), lambda i: (0, 0)),
            pl.BlockSpec((1, d_out), lambda i: (0, 0)),
        ],
        out_specs=pl.BlockSpec((block, d_out), lambda i: (i, 0)),
        out_shape=jax.ShapeDtypeStruct((N_NODES, d_out), jnp.float32),
    )(x, parts, W, b.reshape(1, d_out))


def kernel(x, edge_index, W1, b1, W2, b2):
    src = edge_index[0].astype(jnp.int32).reshape(NW, NSB, SBC, CHUNK)
    dst = edge_index[1].astype(jnp.int32).reshape(NW, NSB, SBC, CHUNK)

    parts1 = _sc_aggregate(x, src, dst)
    h = _tc_mlp(x, parts1, W1, b1, relu=True)
    parts2 = _sc_aggregate(h, src, dst)
    out = _tc_mlp(h, parts2, W2, b2, relu=False)
    return out


# speedup vs baseline: 1.0280x; 1.0280x over previous
import functools

import jax
import jax.numpy as jnp
from jax import lax
from jax.experimental import pallas as pl
from jax.experimental.pallas import tpu as pltpu
from jax.experimental.pallas import tpu_sc as plsc

N_NODES = 10000
N_EDGES = 320000
D = 128

NC = 2
NS = 16
NW = NC * NS

EDGES_PER_W = N_EDGES // NW
CHUNK = 80
NCHUNK = EDGES_PER_W // CHUNK
SBC = 5
NSB = NCHUNK // SBC
NBUF = 3
LEAD = 2
ROWS_PER_TILE = 624
REM_ROW0 = NS * ROWS_PER_TILE
REM_ROWS = N_NODES - REM_ROW0


def _sc_aggregate(x, src, dst):
    mesh = plsc.VectorSubcoreMesh(core_axis_name="c", subcore_axis_name="s")

    @functools.partial(
        pl.kernel,
        mesh=mesh,
        out_type=jax.ShapeDtypeStruct((NC, N_NODES, D), jnp.float32),
        scratch_types=[
            pltpu.VMEM((3, SBC, CHUNK), jnp.int32),
            pltpu.VMEM((3, SBC, CHUNK), jnp.int32),
            pltpu.VMEM((NBUF, CHUNK, D), jnp.float32),
            pltpu.VMEM_SHARED((N_NODES, D), jnp.float32),
            pltpu.SemaphoreType.DMA,
            pltpu.SemaphoreType.DMA,
            pltpu.SemaphoreType.DMA((NBUF,)),
        ],
    )
    def agg_kernel(x_hbm, src_hbm, dst_hbm, out_hbm, src_v, dst_v, rows_v, acc,
                   sem, isem, ssem):
        cid = lax.axis_index("c")
        sid = lax.axis_index("s")
        wid = cid * NS + sid

        def issue_idx_load(s, buf):
            pltpu.async_copy(src_hbm.at[wid, s], src_v.at[buf], isem)
            pltpu.async_copy(dst_hbm.at[wid, s], dst_v.at[buf], isem)

        def wait_idx_load(s, buf):
            pltpu.make_async_copy(src_hbm.at[wid, s], src_v.at[buf], isem).wait()
            pltpu.make_async_copy(dst_hbm.at[wid, s], dst_v.at[buf], isem).wait()

        def issue_gather(sbuf, row, b):
            pltpu.async_copy(x_hbm.at[src_v.at[sbuf].at[row]], rows_v.at[b], sem)

        issue_idx_load(0, 0)
        issue_idx_load(1, 1)

        zeros16 = jnp.zeros((16,), jnp.float32)
        zbuf = rows_v.at[NBUF - 1]

        def zero_row(r, _):
            for c in range(D // 16):
                zbuf[r, pl.ds(c * 16, 16)] = zeros16
            return 0

        lax.fori_loop(0, CHUNK, zero_row, 0)

        wait_idx_load(0, 0)
        for i in range(LEAD):
            issue_gather(0, i, i)

        row0 = sid * ROWS_PER_TILE
        nfull, nrem = divmod(ROWS_PER_TILE, CHUNK)
        for k in range(nfull):
            pltpu.sync_copy(zbuf, acc.at[pl.ds(row0 + k * CHUNK, CHUNK)])
        if nrem:
            pltpu.sync_copy(zbuf.at[pl.ds(0, nrem)],
                            acc.at[pl.ds(row0 + nfull * CHUNK, nrem)])

        @pl.when(sid == NS - 1)
        def _zero_rem():
            pltpu.sync_copy(zbuf.at[pl.ds(0, REM_ROWS)],
                            acc.at[pl.ds(REM_ROW0, REM_ROWS)])

        plsc.subcore_barrier()

        def sb_body(s, _):
            bsb = lax.rem(s, 3)
            bsb_next = lax.rem(s + 1, 3)
            for jj in range(SBC):
                j = s * SBC + jj
                b = lax.rem(j, NBUF)
                pltpu.make_async_copy(x_hbm.at[src_v.at[bsb].at[jj]],
                                      rows_v.at[b], sem).wait()
                pltpu.async_copy(rows_v.at[b], acc.at[dst_v.at[bsb].at[jj]],
                                 ssem.at[b], add=True)

                if jj == SBC - LEAD:
                    @pl.when(s + 1 < NSB)
                    def _wait_next_idx():
                        wait_idx_load(s + 1, bsb_next)

                bnext = lax.rem(j + LEAD, NBUF)

                @pl.when(j + LEAD < NCHUNK)
                def _lookahead_gather():
                    @pl.when(j >= NBUF - LEAD)
                    def _free_buf():
                        pltpu.make_async_copy(
                            rows_v.at[bnext], acc.at[dst_v.at[bsb].at[jj]],
                            ssem.at[bnext]).wait()
                    if jj + LEAD < SBC:
                        issue_gather(bsb, jj + LEAD, bnext)
                    else:
                        issue_gather(bsb_next, jj + LEAD - SBC, bnext)

            @pl.when(s + 2 < NSB)
            def _prefetch_idx():
                issue_idx_load(s + 2, lax.rem(s + 2, 3))

            return 0

        lax.fori_loop(0, NSB, sb_body, 0)

        for b in range(NBUF):
            pltpu.make_async_copy(rows_v.at[b], acc.at[dst_v.at[0].at[0]],
                                  ssem.at[b]).wait()
        plsc.subcore_barrier()

        pltpu.sync_copy(acc.at[pl.ds(row0, ROWS_PER_TILE)],
                        out_hbm.at[cid, pl.ds(row0, ROWS_PER_TILE)])

        @pl.when(sid == NS - 1)
        def _write_rem():
            pltpu.sync_copy(acc.at[pl.ds(REM_ROW0, REM_ROWS)],
                            out_hbm.at[cid, pl.ds(REM_ROW0, REM_ROWS)])

    return agg_kernel(x, src, dst)


def _tc_mlp(x, parts, W, b, relu):
    d_out = W.shape[1]
    block = 2000
    grid = N_NODES // block

    def mlp_kernel(x_ref, p_ref, w_ref, b_ref, o_ref):
        h = x_ref[...] + p_ref[0] + p_ref[1]
        y = jnp.dot(h, w_ref[...], preferred_element_type=jnp.float32) + b_ref[...]
        if relu:
            y = jnp.maximum(y, 0.0)
        o_ref[...] = y

    return pl.pallas_call(
        mlp_kernel,
        grid=(grid,),
        in_specs=[
            pl.BlockSpec((block, D), lambda i: (i, 0)),
            pl.BlockSpec((NC, block, D), lambda i: (0, i, 0)),
            pl.BlockSpec((D, d_out), lambda i: (0, 0)),
            pl.BlockSpec((1, d_out), lambda i: (0, 0)),
        ],
        out_specs=pl.BlockSpec((block, d_out), lambda i: (i, 0)),
        out_shape=jax.ShapeDtypeStruct((N_NODES, d_out), jnp.float32),
    )(x, parts, W, b.reshape(1, d_out))


def kernel(x, edge_index, W1, b1, W2, b2):
    src = edge_index[0].astype(jnp.int32).reshape(NW, NSB, SBC, CHUNK)
    dst = edge_index[1].astype(jnp.int32).reshape(NW, NSB, SBC, CHUNK)

    parts1 = _sc_aggregate(x, src, dst)
    h = _tc_mlp(x, parts1, W1, b1, relu=True)
    parts2 = _sc_aggregate(h, src, dst)
    out = _tc_mlp(h, parts2, W2, b2, relu=False)
    return out


# speedup vs baseline: 1.0348x; 1.0066x over previous
import functools

import jax
import jax.numpy as jnp
from jax import lax
from jax.experimental import pallas as pl
from jax.experimental.pallas import tpu as pltpu
from jax.experimental.pallas import tpu_sc as plsc

N_NODES = 10000
N_EDGES = 320000
D = 128

NC = 2
NS = 16
NW = NC * NS

EDGES_PER_W = N_EDGES // NW
CHUNK = 80
NCHUNK = EDGES_PER_W // CHUNK
SBC = 5
NSB = NCHUNK // SBC
NBUF = 3
LEAD = 2
ROWS_PER_TILE = 624
REM_ROW0 = NS * ROWS_PER_TILE
REM_ROWS = N_NODES - REM_ROW0


def _sc_aggregate(x, src, dst):
    mesh = plsc.VectorSubcoreMesh(core_axis_name="c", subcore_axis_name="s")

    @functools.partial(
        pl.kernel,
        mesh=mesh,
        out_type=jax.ShapeDtypeStruct((NC, N_NODES, D), jnp.float32),
        scratch_types=[
            pltpu.VMEM((3, SBC, CHUNK), jnp.int32),
            pltpu.VMEM((3, SBC, CHUNK), jnp.int32),
            pltpu.VMEM((NBUF, CHUNK, D), jnp.float32),
            pltpu.VMEM_SHARED((N_NODES, D), jnp.float32),
            pltpu.SemaphoreType.DMA,
            pltpu.SemaphoreType.DMA,
            pltpu.SemaphoreType.DMA((NBUF,)),
        ],
    )
    def agg_kernel(x_hbm, src_hbm, dst_hbm, out_hbm, src_v, dst_v, rows_v, acc,
                   sem, isem, ssem):
        cid = lax.axis_index("c")
        sid = lax.axis_index("s")
        wid = cid * NS + sid

        def issue_idx_load(s, buf):
            pltpu.async_copy(src_hbm.at[wid, s], src_v.at[buf], isem)
            pltpu.async_copy(dst_hbm.at[wid, s], dst_v.at[buf], isem)

        def wait_idx_load(s, buf):
            pltpu.make_async_copy(src_hbm.at[wid, s], src_v.at[buf], isem).wait()
            pltpu.make_async_copy(dst_hbm.at[wid, s], dst_v.at[buf], isem).wait()

        def issue_gather(sbuf, row, b):
            pltpu.async_copy(x_hbm.at[src_v.at[sbuf].at[row]], rows_v.at[b], sem)

        issue_idx_load(0, 0)
        issue_idx_load(1, 1)

        zeros16 = jnp.zeros((16,), jnp.float32)
        zbuf = rows_v.at[NBUF - 1]

        def zero_row(r, _):
            for c in range(D // 16):
                zbuf[r, pl.ds(c * 16, 16)] = zeros16
            return 0

        lax.fori_loop(0, CHUNK, zero_row, 0)

        wait_idx_load(0, 0)
        for i in range(LEAD):
            issue_gather(0, i, i)

        row0 = sid * ROWS_PER_TILE
        nfull, nrem = divmod(ROWS_PER_TILE, CHUNK)
        for k in range(nfull):
            pltpu.sync_copy(zbuf, acc.at[pl.ds(row0 + k * CHUNK, CHUNK)])
        if nrem:
            pltpu.sync_copy(zbuf.at[pl.ds(0, nrem)],
                            acc.at[pl.ds(row0 + nfull * CHUNK, nrem)])

        @pl.when(sid == NS - 1)
        def _zero_rem():
            pltpu.sync_copy(zbuf.at[pl.ds(0, REM_ROWS)],
                            acc.at[pl.ds(REM_ROW0, REM_ROWS)])

        plsc.subcore_barrier()

        def sb_body(s, _):
            bsb = lax.rem(s, 3)
            bsb_next = lax.rem(s + 1, 3)
            for jj in range(SBC):
                j = s * SBC + jj
                b = lax.rem(j, NBUF)
                pltpu.make_async_copy(x_hbm.at[src_v.at[bsb].at[jj]],
                                      rows_v.at[b], sem).wait()
                pltpu.async_copy(rows_v.at[b], acc.at[dst_v.at[bsb].at[jj]],
                                 ssem.at[b], add=True)

                if jj == SBC - LEAD:
                    @pl.when(s + 1 < NSB)
                    def _wait_next_idx():
                        wait_idx_load(s + 1, bsb_next)

                bnext = lax.rem(j + LEAD, NBUF)

                @pl.when(j + LEAD < NCHUNK)
                def _lookahead_gather():
                    @pl.when(j >= NBUF - LEAD)
                    def _free_buf():
                        pltpu.make_async_copy(
                            rows_v.at[bnext], acc.at[dst_v.at[bsb].at[jj]],
                            ssem.at[bnext]).wait()
                    if jj + LEAD < SBC:
                        issue_gather(bsb, jj + LEAD, bnext)
                    else:
                        issue_gather(bsb_next, jj + LEAD - SBC, bnext)

            @pl.when(s + 2 < NSB)
            def _prefetch_idx():
                issue_idx_load(s + 2, lax.rem(s + 2, 3))

            return 0

        lax.fori_loop(0, NSB, sb_body, 0)

        for b in range(NBUF):
            pltpu.make_async_copy(rows_v.at[b], acc.at[dst_v.at[0].at[0]],
                                  ssem.at[b]).wait()
        plsc.subcore_barrier()

        pltpu.sync_copy(acc.at[pl.ds(row0, ROWS_PER_TILE)],
                        out_hbm.at[cid, pl.ds(row0, ROWS_PER_TILE)])

        @pl.when(sid == NS - 1)
        def _write_rem():
            pltpu.sync_copy(acc.at[pl.ds(REM_ROW0, REM_ROWS)],
                            out_hbm.at[cid, pl.ds(REM_ROW0, REM_ROWS)])

    return agg_kernel(x, src, dst)


def _tc_mlp(x, parts, W, b, relu):
    d_out = W.shape[1]
    block = 5000
    grid = N_NODES // block

    def mlp_kernel(x_ref, p_ref, w_ref, b_ref, o_ref):
        h = x_ref[...] + p_ref[0] + p_ref[1]
        y = jnp.dot(h, w_ref[...], preferred_element_type=jnp.float32) + b_ref[...]
        if relu:
            y = jnp.maximum(y, 0.0)
        o_ref[...] = y

    return pl.pallas_call(
        mlp_kernel,
        grid=(grid,),
        in_specs=[
            pl.BlockSpec((block, D), lambda i: (i, 0)),
            pl.BlockSpec((NC, block, D), lambda i: (0, i, 0)),
            pl.BlockSpec((D, d_out), lambda i: (0, 0)),
            pl.BlockSpec((1, d_out), lambda i: (0, 0)),
        ],
        out_specs=pl.BlockSpec((block, d_out), lambda i: (i, 0)),
        out_shape=jax.ShapeDtypeStruct((N_NODES, d_out), jnp.float32),
    )(x, parts, W, b.reshape(1, d_out))


def kernel(x, edge_index, W1, b1, W2, b2):
    src = edge_index[0].astype(jnp.int32).reshape(NW, NSB, SBC, CHUNK)
    dst = edge_index[1].astype(jnp.int32).reshape(NW, NSB, SBC, CHUNK)

    parts1 = _sc_aggregate(x, src, dst)
    h = _tc_mlp(x, parts1, W1, b1, relu=True)
    parts2 = _sc_aggregate(h, src, dst)
    out = _tc_mlp(h, parts2, W2, b2, relu=False)
    return out
